# baseline (speedup 1.0000x reference)
import functools

import jax
import jax.numpy as jnp
from jax import lax
from jax.experimental import pallas as pl
from jax.experimental.pallas import tpu as pltpu
from jax.experimental.pallas import tpu_sc as plsc

_D = 1000
_N = 51200
_NW = 32
_RPW = _N // _NW
_CHUNK = 32
_NCHUNK = _RPW // _CHUNK


_B, _T = 1024, 50
_BPW = _B // _NW


@functools.partial(
    pl.kernel,
    out_type=jax.ShapeDtypeStruct((_B, _T, _D), jnp.float32),
    mesh=plsc.VectorSubcoreMesh(core_axis_name="c", subcore_axis_name="s"),
    compiler_params=pltpu.CompilerParams(use_tc_tiling_on_sc=False),
    scratch_types=[
        pltpu.VMEM((_BPW * 56,), jnp.int32),
        pltpu.VMEM((_T, _D), jnp.float32),
        pltpu.VMEM_SHARED((1000, _D), jnp.float32),
        pltpu.SemaphoreType.DMA,
        pltpu.SemaphoreType.DMA,
    ],
)
def _sc_gather(table_hbm, idx_hbm, out_hbm, idx_v, buf, tab_sp, gsem, wsem):
    sid = lax.axis_index("s")
    wid = sid * 2 + lax.axis_index("c")

    @pl.when(sid == 0)
    def _():
        pltpu.sync_copy(table_hbm, tab_sp)

    pltpu.sync_copy(idx_hbm.at[pl.ds(wid * _BPW * 56, _BPW * 56)], idx_v)
    plsc.subcore_barrier()

    def body(j, carry):
        pltpu.async_copy(
            tab_sp.at[idx_v.at[pl.ds(j * 56, _T)]], buf, gsem
        ).wait()
        pltpu.async_copy(buf, out_hbm.at[wid * _BPW + j], wsem).wait()
        return carry

    lax.fori_loop(0, _BPW, body, 0)


def kernel(idx, token_embedding_table):
    idx_p = jnp.pad(idx, ((0, 0), (0, 6))).reshape(-1)
    return _sc_gather(token_embedding_table, idx_p)
